# baseline (speedup 1.0000x reference)
import functools

import jax
import jax.numpy as jnp
from jax import lax
from jax.experimental import pallas as pl
from jax.experimental.pallas import tpu as pltpu
from jax.experimental.pallas import tpu_sc as plsc

EMB = 64
SCALE = 8.0
VOCAB = 1000000

NC = 2
NS = 16
NW = NC * NS
JT, JL = 25, 8
TI, IL = 32, 128
TK, KL = 8, 8
NUNIT = JT * JL

NBLK = VOCAB // 128
BLK_PER_W = (NBLK + NW - 1) // NW


def _fmt_body(tabT, tail2, tab2, blk0, blk1, vt0, vt1,
              rsem0, rsem1, wsem0, wsem1):
    w = lax.axis_index("s") * NC + lax.axis_index("c")
    blk = (blk0, blk1)
    vt = (vt0, vt1)
    rsem = (rsem0, rsem1)
    wsem = (wsem0, wsem1)
    iota = lax.iota(jnp.int32, 16)

    def fire_read(n, b):
        c = n * NW + w
        pltpu.async_copy(tabT.at[:, pl.ds(c * 128, 128)], blk[b], rsem[b])

    def transpose_block(b):
        pltpu.make_async_copy(tabT.at[:, pl.ds(0, 128)], blk[b], rsem[b]).wait()

        for g in range(8):
            row16 = iota + (g % 4) * 16
            off_c = g // 4
            sl = pl.ds(g * 16, 16)

            @plsc.parallel_loop(0, 64, unroll=4)
            def _(m):
                col16 = iota * 0 + (2 * m + off_c)
                vt[b][m, sl] = plsc.load_gather(blk[b], [row16, col16])

    def fire_write(n, b):
        c = n * NW + w
        pltpu.async_copy(vt[b], tab2.at[pl.ds(c * 64, 64)], wsem[b])

    def drain_write(b):
        pltpu.make_async_copy(vt[b], tab2.at[pl.ds(0, 64)], wsem[b]).wait()

    fire_read(0, 0)

    def pair(p, carry):
        for b in range(2):
            n = p * 2 + b

            @pl.when((n >= 2) & ((n - 2) * NW + w < NBLK))
            def _():
                drain_write(b)

            @pl.when(n * NW + w < NBLK)
            def _():
                @pl.when((n + 1) * NW + w < NBLK)
                def _():
                    fire_read(n + 1, 1 - b)

                transpose_block(b)
                fire_write(n, b)
        return carry

    lax.fori_loop(0, (BLK_PER_W + 1) // 2, pair, 0)

    @pl.when((BLK_PER_W - 1) * NW + w < NBLK)
    def _():
        drain_write((BLK_PER_W - 1) % 2)

    @pl.when(w == 0)
    def _():
        pltpu.sync_copy(tail2, vt0.at[pl.ds(0, 32)])
        pltpu.sync_copy(vt0.at[pl.ds(0, 32)], tab2.at[pl.ds(NBLK * 64, 32)])


@functools.partial(
    pl.kernel,
    mesh=plsc.VectorSubcoreMesh(core_axis_name="c", subcore_axis_name="s"),
    out_type=jax.ShapeDtypeStruct((VOCAB // 2, 128), jnp.float32),
    scratch_types=[
        pltpu.VMEM((EMB, 128), jnp.float32),
        pltpu.VMEM((EMB, 128), jnp.float32),
        pltpu.VMEM((EMB, 128), jnp.float32),
        pltpu.VMEM((EMB, 128), jnp.float32),
        pltpu.SemaphoreType.DMA,
        pltpu.SemaphoreType.DMA,
        pltpu.SemaphoreType.DMA,
        pltpu.SemaphoreType.DMA,
    ],
    compiler_params=pltpu.CompilerParams(needs_layout_passes=False),
)
def _fmt_kernel(tabT, tail2, tab2, *scratch):
    _fmt_body(tabT, tail2, tab2, *scratch)


def _emb_body(tok4, tab2, out5,
              tokv0, tokv1, idx0, idx1, par0, par1,
              rows0, rows1, outv0, outv1,
              tsem0, tsem1, gsem0, gsem1, wsem0, wsem1):
    w = lax.axis_index("s") * NC + lax.axis_index("c")
    tokv = (tokv0, tokv1)
    idx = (idx0, idx1)
    par = (par0, par1)
    rows = (rows0, rows1)
    outv = (outv0, outv1)
    tsem = (tsem0, tsem1)
    gsem = (gsem0, gsem1)
    wsem = (wsem0, wsem1)
    iota = lax.iota(jnp.int32, 16)

    def tok_fetch(u, b):
        jt = u // JL
        jl = u % JL
        pltpu.async_copy(tok4.at[jt, w, jl], tokv[b], tsem[b])

    def fire_gather(b):
        pltpu.make_async_copy(tok4.at[0, 0, 0], tokv[b], tsem[b]).wait()
        for g in range(8):
            sl = pl.ds(g * 16, 16)
            t16 = tokv[b][sl]
            idx[b][sl] = t16 >> 1
            par[b][sl] = (t16 & 1) << 6
        pltpu.async_copy(tab2.at[idx[b]], rows[b], gsem[b])

    def compute(u, b):
        pltpu.make_async_copy(tab2.at[pl.ds(0, IL)], rows[b], gsem[b]).wait()
        for g in range(8):
            sl = pl.ds(g * 16, 16)
            row16 = iota + (g * 16)
            p16 = par[b][sl]

            @plsc.parallel_loop(0, TK * KL, unroll=4)
            def _(tkkl):
                col16 = p16 + tkkl
                v = plsc.load_gather(rows[b], [row16, col16])
                outv[b][tkkl // KL, tkkl % KL, sl] = v * SCALE

    def fire_write(u, b):
        for tk in range(TK):
            pltpu.async_copy(outv[b].at[tk], out5.at[u, tk, w], wsem[b])

    def drain_write(b):
        for tk in range(TK):
            pltpu.make_async_copy(
                outv[b].at[tk], out5.at[0, tk, 0], wsem[b]).wait()

    tok_fetch(0, 0)
    tok_fetch(1, 1)
    fire_gather(0)

    def pair(p, carry):
        for b in range(2):
            u = p * 2 + b

            @pl.when(u + 1 < NUNIT)
            def _():
                fire_gather(1 - b)

            @pl.when(u + 2 < NUNIT)
            def _():
                tok_fetch(u + 2, b)

            @pl.when(u >= 2)
            def _():
                drain_write(b)

            compute(u, b)
            fire_write(u, b)
        return carry

    lax.fori_loop(0, NUNIT // 2, pair, 0)
    drain_write(0)
    drain_write(1)


@functools.partial(
    pl.kernel,
    mesh=plsc.VectorSubcoreMesh(core_axis_name="c", subcore_axis_name="s"),
    out_type=jax.ShapeDtypeStruct((JT * JL, TK, TI, KL, IL), jnp.float32),
    scratch_types=[
        pltpu.VMEM((IL,), jnp.int32),
        pltpu.VMEM((IL,), jnp.int32),
        pltpu.VMEM((IL,), jnp.int32),
        pltpu.VMEM((IL,), jnp.int32),
        pltpu.VMEM((IL,), jnp.int32),
        pltpu.VMEM((IL,), jnp.int32),
        pltpu.VMEM((IL, 128), jnp.float32),
        pltpu.VMEM((IL, 128), jnp.float32),
        pltpu.VMEM((TK, KL, IL), jnp.float32),
        pltpu.VMEM((TK, KL, IL), jnp.float32),
        pltpu.SemaphoreType.DMA,
        pltpu.SemaphoreType.DMA,
        pltpu.SemaphoreType.DMA,
        pltpu.SemaphoreType.DMA,
        pltpu.SemaphoreType.DMA,
        pltpu.SemaphoreType.DMA,
    ],
    compiler_params=pltpu.CompilerParams(needs_layout_passes=False),
)
def _emb_kernel(tok4, tab2, out5, *scratch):
    _emb_body(tok4, tab2, out5, *scratch)


def kernel(tokens, table):
    tok4 = tokens.astype(jnp.int32).reshape(TI, IL, JT, JL).transpose(2, 0, 3, 1)
    tabT = table.T
    tail2 = table[NBLK * 128:].reshape(32, 128)
    tab2 = _fmt_kernel(tabT, tail2)
    out5 = _emb_kernel(tok4, tab2)
    out = out5.transpose(2, 4, 0, 1, 3).reshape(4096, JT * JL, EMB)
    return out


# speedup vs baseline: 1.0280x; 1.0280x over previous
import functools

import jax
import jax.numpy as jnp
from jax import lax
from jax.experimental import pallas as pl
from jax.experimental.pallas import tpu as pltpu
from jax.experimental.pallas import tpu_sc as plsc

EMB = 64
SCALE = 8.0
VOCAB = 1000000

NC = 2
NS = 16
NW = NC * NS
JT, JL = 25, 8
TI, IL = 32, 128
TK, KL = 8, 8
NUNIT = JT * JL

NBLK = VOCAB // 128
BLK_PER_W = (NBLK + NW - 1) // NW


def _fmt_body(tabT, tail2, tab2, blk0, blk1, vt0, vt1,
              rsem0, rsem1, wsem0, wsem1):
    w = lax.axis_index("s") * NC + lax.axis_index("c")
    blk = (blk0, blk1)
    vt = (vt0, vt1)
    rsem = (rsem0, rsem1)
    wsem = (wsem0, wsem1)
    iota = lax.iota(jnp.int32, 16)

    def fire_read(n, b):
        c = n * NW + w
        pltpu.async_copy(tabT.at[:, pl.ds(c * 128, 128)], blk[b], rsem[b])

    def transpose_block(b):
        pltpu.make_async_copy(tabT.at[:, pl.ds(0, 128)], blk[b], rsem[b]).wait()
        rowv = [iota + q * 16 for q in range(4)]
        z16 = iota * 0

        @plsc.parallel_loop(0, 64, unroll=4)
        def _(m):
            c0 = z16 + 2 * m
            c1 = z16 + (2 * m + 1)
            for g in range(8):
                col16 = c0 if g < 4 else c1
                vt[b][m, pl.ds(g * 16, 16)] = plsc.load_gather(
                    blk[b], [rowv[g % 4], col16])

    def fire_write(n, b):
        c = n * NW + w
        pltpu.async_copy(vt[b], tab2.at[pl.ds(c * 64, 64)], wsem[b])

    def drain_write(b):
        pltpu.make_async_copy(vt[b], tab2.at[pl.ds(0, 64)], wsem[b]).wait()

    fire_read(0, 0)

    def pair(p, carry):
        for b in range(2):
            n = p * 2 + b

            @pl.when((n >= 2) & ((n - 2) * NW + w < NBLK))
            def _():
                drain_write(b)

            @pl.when(n * NW + w < NBLK)
            def _():
                @pl.when((n + 1) * NW + w < NBLK)
                def _():
                    fire_read(n + 1, 1 - b)

                transpose_block(b)
                fire_write(n, b)
        return carry

    lax.fori_loop(0, (BLK_PER_W + 1) // 2, pair, 0)

    @pl.when((BLK_PER_W - 1) * NW + w < NBLK)
    def _():
        drain_write((BLK_PER_W - 1) % 2)

    @pl.when(w == 0)
    def _():
        pltpu.sync_copy(tail2, vt0.at[pl.ds(0, 32)])
        pltpu.sync_copy(vt0.at[pl.ds(0, 32)], tab2.at[pl.ds(NBLK * 64, 32)])


@functools.partial(
    pl.kernel,
    mesh=plsc.VectorSubcoreMesh(core_axis_name="c", subcore_axis_name="s"),
    out_type=jax.ShapeDtypeStruct((VOCAB // 2, 128), jnp.float32),
    scratch_types=[
        pltpu.VMEM((EMB, 128), jnp.float32),
        pltpu.VMEM((EMB, 128), jnp.float32),
        pltpu.VMEM((EMB, 128), jnp.float32),
        pltpu.VMEM((EMB, 128), jnp.float32),
        pltpu.SemaphoreType.DMA,
        pltpu.SemaphoreType.DMA,
        pltpu.SemaphoreType.DMA,
        pltpu.SemaphoreType.DMA,
    ],
    compiler_params=pltpu.CompilerParams(needs_layout_passes=False),
)
def _fmt_kernel(tabT, tail2, tab2, *scratch):
    _fmt_body(tabT, tail2, tab2, *scratch)


def _emb_body(tok4, tab2, out5,
              tokv0, tokv1, idx0, idx1, par0, par1,
              rows0, rows1, outv0, outv1,
              tsem0, tsem1, gsem0, gsem1, wsem0, wsem1):
    w = lax.axis_index("s") * NC + lax.axis_index("c")
    tokv = (tokv0, tokv1)
    idx = (idx0, idx1)
    par = (par0, par1)
    rows = (rows0, rows1)
    outv = (outv0, outv1)
    tsem = (tsem0, tsem1)
    gsem = (gsem0, gsem1)
    wsem = (wsem0, wsem1)
    iota = lax.iota(jnp.int32, 16)

    def tok_fetch(u, b):
        jt = u // JL
        jl = u % JL
        pltpu.async_copy(tok4.at[jt, w, jl], tokv[b], tsem[b])

    def fire_gather(b):
        pltpu.make_async_copy(tok4.at[0, 0, 0], tokv[b], tsem[b]).wait()
        for g in range(8):
            sl = pl.ds(g * 16, 16)
            t16 = tokv[b][sl]
            idx[b][sl] = t16 >> 1
            par[b][sl] = (t16 & 1) << 6
        pltpu.async_copy(tab2.at[idx[b]], rows[b], gsem[b])

    def compute(u, b):
        pltpu.make_async_copy(tab2.at[pl.ds(0, IL)], rows[b], gsem[b]).wait()
        rowv = [iota + g * 16 for g in range(8)]
        parv = [par[b][pl.ds(g * 16, 16)] for g in range(8)]

        @plsc.parallel_loop(0, TK * KL, unroll=4)
        def _(k):
            for g in range(8):
                v = plsc.load_gather(rows[b], [rowv[g], parv[g] + k])
                outv[b][k, pl.ds(g * 16, 16)] = v * SCALE

    def fire_write(u, b):
        for tk in range(TK):
            pltpu.async_copy(
                outv[b].at[pl.ds(tk * KL, KL)], out5.at[u, tk, w], wsem[b])

    def drain_write(b):
        for tk in range(TK):
            pltpu.make_async_copy(
                outv[b].at[pl.ds(tk * KL, KL)], out5.at[0, tk, 0],
                wsem[b]).wait()

    tok_fetch(0, 0)
    tok_fetch(1, 1)
    fire_gather(0)

    def pair(p, carry):
        for b in range(2):
            u = p * 2 + b

            @pl.when(u + 1 < NUNIT)
            def _():
                fire_gather(1 - b)

            @pl.when(u + 2 < NUNIT)
            def _():
                tok_fetch(u + 2, b)

            @pl.when(u >= 2)
            def _():
                drain_write(b)

            compute(u, b)
            fire_write(u, b)
        return carry

    lax.fori_loop(0, NUNIT // 2, pair, 0)
    drain_write(0)
    drain_write(1)


@functools.partial(
    pl.kernel,
    mesh=plsc.VectorSubcoreMesh(core_axis_name="c", subcore_axis_name="s"),
    out_type=jax.ShapeDtypeStruct((JT * JL, TK, TI, KL, IL), jnp.float32),
    scratch_types=[
        pltpu.VMEM((IL,), jnp.int32),
        pltpu.VMEM((IL,), jnp.int32),
        pltpu.VMEM((IL,), jnp.int32),
        pltpu.VMEM((IL,), jnp.int32),
        pltpu.VMEM((IL,), jnp.int32),
        pltpu.VMEM((IL,), jnp.int32),
        pltpu.VMEM((IL, 128), jnp.float32),
        pltpu.VMEM((IL, 128), jnp.float32),
        pltpu.VMEM((TK * KL, IL), jnp.float32),
        pltpu.VMEM((TK * KL, IL), jnp.float32),
        pltpu.SemaphoreType.DMA,
        pltpu.SemaphoreType.DMA,
        pltpu.SemaphoreType.DMA,
        pltpu.SemaphoreType.DMA,
        pltpu.SemaphoreType.DMA,
        pltpu.SemaphoreType.DMA,
    ],
    compiler_params=pltpu.CompilerParams(needs_layout_passes=False),
)
def _emb_kernel(tok4, tab2, out5, *scratch):
    _emb_body(tok4, tab2, out5, *scratch)


def kernel(tokens, table):
    tok4 = tokens.astype(jnp.int32).reshape(TI, IL, JT, JL).transpose(2, 0, 3, 1)
    tabT = table.T
    tail2 = table[NBLK * 128:].reshape(32, 128)
    tab2 = _fmt_kernel(tabT, tail2)
    out5 = _emb_kernel(tok4, tab2)
    out = out5.transpose(2, 4, 0, 1, 3).reshape(4096, JT * JL, EMB)
    return out
